# baseline (speedup 1.0000x reference)
import functools
import jax
import jax.numpy as jnp
from jax import lax
from jax.experimental import pallas as pl
from jax.experimental.pallas import tpu as pltpu
from jax.experimental.pallas import tpu_sc as plsc

_NC = 2
_NS = 16


def _magic_div(ncol, pmax):
    for shift in range(16, 31):
        mul = (1 << shift) // ncol + 1
        if all((p * mul) >> shift == p // ncol for p in range(pmax)):
            return mul, shift
    raise ValueError(f"no exact multiply-shift division for {ncol=}, {pmax=}")


def _linearize_body(rows_per_w, b_per_w, ncol, div_mul, div_shift,
                    xt_hbm, idx_hbm, stage_v, idx_v, sem):
    wid = lax.axis_index("s") * _NC + lax.axis_index("c")
    pltpu.sync_copy(xt_hbm.at[:, pl.ds(wid * rows_per_w, rows_per_w)], stage_v)
    lane = lax.iota(jnp.int32, 16)

    def linearize(k, carry):
        p = k * 16 + lane
        r = lax.shift_right_logical(p * div_mul, div_shift)
        c = p - r * ncol
        idx_v[pl.ds(k * 16, 16)] = plsc.load_gather(stage_v, [c, r])
        return carry

    lax.fori_loop(0, b_per_w // 16, linearize, 0, unroll=8)
    pltpu.sync_copy(idx_v, idx_hbm.at[pl.ds(wid * b_per_w, b_per_w)])


def _row_gather_body(b_per_w, chunk, table_hbm, idx_hbm, out_hbm,
                     idx_v, row_v, rows8_v, out_v, sem):
    wid = lax.axis_index("s") * _NC + lax.axis_index("c")
    base = wid * b_per_w
    lane = lax.iota(jnp.int32, 16)

    def do_chunk(ci, carry):
        pltpu.sync_copy(idx_hbm.at[pl.ds(base + ci * chunk, chunk)], idx_v)

        def to_rows(t, carry2):
            idx16 = idx_v[pl.ds(t * 16, 16)]
            row_v[pl.ds(t * 16, 16)] = lax.shift_right_logical(idx16, 3)
            return carry2

        lax.fori_loop(0, chunk // 16, to_rows, 0, unroll=8)
        pltpu.async_copy(table_hbm.at[row_v], rows8_v, sem).wait()

        def select(t, carry2):
            idx16 = idx_v[pl.ds(t * 16, 16)]
            sub = lax.bitwise_and(idx16, 7)
            i16 = t * 16 + lane
            out_v[pl.ds(t * 16, 16)] = plsc.load_gather(rows8_v, [i16, sub])
            return carry2

        lax.fori_loop(0, chunk // 16, select, 0, unroll=8)
        pltpu.sync_copy(out_v, out_hbm.at[pl.ds(base + ci * chunk, chunk)])
        return carry

    lax.fori_loop(0, b_per_w // chunk, do_chunk, 0)


def kernel(x, emb_weight):
    B = x.shape[0] * x.shape[1]
    V = emb_weight.shape[0]
    nw = _NC * _NS
    b_per_w = B // nw
    rows_per_w = x.shape[0] // nw
    ncol = x.shape[1]
    assert B % nw == 0 and b_per_w % 8 == 0 and V % 8 == 0

    idx = x.astype(jnp.int32)
    div_mul, div_shift = _magic_div(ncol, b_per_w)
    mesh = plsc.VectorSubcoreMesh(core_axis_name="c", subcore_axis_name="s")

    linearize = pl.kernel(
        functools.partial(_linearize_body, rows_per_w, b_per_w, ncol,
                          div_mul, div_shift),
        mesh=mesh,
        out_type=jax.ShapeDtypeStruct((B,), jnp.int32),
        scratch_types=[
            pltpu.VMEM((ncol, rows_per_w), jnp.int32),
            pltpu.VMEM((b_per_w,), jnp.int32),
            pltpu.SemaphoreType.DMA,
        ],
        compiler_params=pltpu.CompilerParams(needs_layout_passes=False),
    )

    chunk = b_per_w // 4
    row_gather = pl.kernel(
        functools.partial(_row_gather_body, b_per_w, chunk),
        mesh=mesh,
        out_type=jax.ShapeDtypeStruct((B,), jnp.float32),
        scratch_types=[
            pltpu.VMEM((chunk,), jnp.int32),
            pltpu.VMEM((chunk,), jnp.int32),
            pltpu.VMEM((chunk, 8), jnp.float32),
            pltpu.VMEM((chunk,), jnp.float32),
            pltpu.SemaphoreType.DMA,
        ],
        compiler_params=pltpu.CompilerParams(
            use_tc_tiling_on_sc=False, needs_layout_passes=False
        ),
    )

    idx_flat = linearize(idx.T)
    out = row_gather(emb_weight.reshape(V // 8, 8), idx_flat)
    return out.reshape(-1, emb_weight.shape[1])


# speedup vs baseline: 1.2140x; 1.2140x over previous
import functools
import jax
import jax.numpy as jnp
from jax import lax
from jax.experimental import pallas as pl
from jax.experimental.pallas import tpu as pltpu
from jax.experimental.pallas import tpu_sc as plsc

_NC = 2
_NS = 16


def _magic_div(ncol, pmax):
    for shift in range(16, 31):
        mul = (1 << shift) // ncol + 1
        if all((p * mul) >> shift == p // ncol for p in range(pmax)):
            return mul, shift
    raise ValueError(f"no exact multiply-shift division for {ncol=}, {pmax=}")


def _linearize_body(rows_per_w, b_per_w, ncol, div_mul, div_shift,
                    xt_hbm, idx_hbm, stage_v, idx_v, sem):
    wid = lax.axis_index("s") * _NC + lax.axis_index("c")
    pltpu.sync_copy(xt_hbm.at[:, pl.ds(wid * rows_per_w, rows_per_w)], stage_v)
    lane = lax.iota(jnp.int32, 16)

    def linearize(k, carry):
        p = k * 16 + lane
        r = lax.shift_right_logical(p * div_mul, div_shift)
        c = p - r * ncol
        idx_v[pl.ds(k * 16, 16)] = plsc.load_gather(stage_v, [c, r])
        return carry

    lax.fori_loop(0, b_per_w // 16, linearize, 0, unroll=8)
    pltpu.sync_copy(idx_v, idx_hbm.at[pl.ds(wid * b_per_w, b_per_w)])


def _flat_gather_body(b_per_w, table_hbm, idx_hbm, out_hbm,
                      idx_v, rows_v, sem):
    wid = lax.axis_index("s") * _NC + lax.axis_index("c")
    base = wid * b_per_w
    pltpu.sync_copy(idx_hbm.at[pl.ds(base, b_per_w)], idx_v)
    pltpu.async_copy(table_hbm.at[idx_v], rows_v, sem).wait()
    pltpu.sync_copy(rows_v, out_hbm.at[pl.ds(base, b_per_w)])


def kernel(x, emb_weight):
    B = x.shape[0] * x.shape[1]
    V = emb_weight.shape[0]
    nw = _NC * _NS
    b_per_w = B // nw
    rows_per_w = x.shape[0] // nw
    ncol = x.shape[1]
    assert B % nw == 0 and b_per_w % 8 == 0 and V % 8 == 0

    idx = x.astype(jnp.int32)
    div_mul, div_shift = _magic_div(ncol, b_per_w)
    mesh = plsc.VectorSubcoreMesh(core_axis_name="c", subcore_axis_name="s")

    linearize = pl.kernel(
        functools.partial(_linearize_body, rows_per_w, b_per_w, ncol,
                          div_mul, div_shift),
        mesh=mesh,
        out_type=jax.ShapeDtypeStruct((B,), jnp.int32),
        scratch_types=[
            pltpu.VMEM((ncol, rows_per_w), jnp.int32),
            pltpu.VMEM((b_per_w,), jnp.int32),
            pltpu.SemaphoreType.DMA,
        ],
        compiler_params=pltpu.CompilerParams(needs_layout_passes=False),
    )

    flat_gather = pl.kernel(
        functools.partial(_flat_gather_body, b_per_w),
        mesh=mesh,
        out_type=jax.ShapeDtypeStruct((B,), jnp.float32),
        scratch_types=[
            pltpu.VMEM((b_per_w,), jnp.int32),
            pltpu.VMEM((b_per_w,), jnp.float32),
            pltpu.SemaphoreType.DMA,
        ],
        compiler_params=pltpu.CompilerParams(needs_layout_passes=False),
    )

    idx_flat = linearize(idx.T)
    out = flat_gather(emb_weight.reshape(-1), idx_flat)
    return out.reshape(-1, emb_weight.shape[1])
